# Initial kernel scaffold; baseline (speedup 1.0000x reference)
import jax
import jax.numpy as jnp
from jax.experimental import pallas as pl


def kernel(node_feats, edge_index, edge_feats, weight, bias, W_src, b_src, W_dst, b_dst, W_edge, b_edge):
    raise NotImplementedError("write your pallas kernel here")


# speedup vs baseline: 1.2514x; 1.2514x over previous
import functools

import jax
import jax.numpy as jnp
from jax import lax
from jax.experimental import pallas as pl
from jax.experimental.pallas import tpu as pltpu
from jax.experimental.pallas import tpu_sc as plsc

N = 10000
E = 320000
D = 128

NC = 2
NS = 16
NW = NC * NS
NPAD = 10240
ROWS_PER_TILE = NPAD // NS
EPW = E // NW
CB = 80
NCHUNK = EPW // CB

_mesh = plsc.VectorSubcoreMesh(
    core_axis_name="c", subcore_axis_name="s", num_cores=NC, num_subcores=NS)


@functools.partial(
    pl.kernel,
    out_type=jax.ShapeDtypeStruct((NW, 2 * NPAD), jnp.float32),
    mesh=_mesh,
    scratch_types=[
        pltpu.VMEM((2 * NPAD,), jnp.float32),
        pltpu.VMEM((EPW,), jnp.int32),
        pltpu.VMEM((EPW,), jnp.int32),
    ],
    compiler_params=pltpu.CompilerParams(needs_layout_passes=False),
)
def _degree_kernel(src_hbm, dst_hbm, deg_out, hist, src_v, dst_v):
    cid = lax.axis_index("c")
    sid = lax.axis_index("s")
    wid = sid * NC + cid

    def zero_body(i, _):
        hist[pl.ds(i * 16, 16)] = jnp.zeros((16,), jnp.float32)
        return _
    lax.fori_loop(0, 2 * NPAD // 16, zero_body, 0, unroll=8)

    base = wid * EPW
    pltpu.sync_copy(src_hbm.at[pl.ds(base, EPW)], src_v)
    pltpu.sync_copy(dst_hbm.at[pl.ds(base, EPW)], dst_v)

    ones = jnp.ones((16,), jnp.float32)

    def edge_body(i, _):
        s = src_v[pl.ds(i * 16, 16)]
        d = dst_v[pl.ds(i * 16, 16)]
        plsc.addupdate_scatter(hist, [s * 2], ones)
        plsc.addupdate_scatter(hist, [d * 2 + 1], ones)
        return _
    lax.fori_loop(0, EPW // 16, edge_body, 0, unroll=4)

    pltpu.sync_copy(hist, deg_out.at[wid])


@functools.partial(
    pl.kernel,
    out_type=[
        jax.ShapeDtypeStruct((E, D), jnp.float32),
        jax.ShapeDtypeStruct((NC, NPAD, D), jnp.float32),
    ],
    mesh=_mesh,
    scratch_types=[
        pltpu.VMEM((CB,), jnp.int32),
        pltpu.VMEM((CB,), jnp.int32),
        pltpu.VMEM((CB, 2 * D), jnp.float32),
        pltpu.VMEM((CB, D), jnp.float32),
        pltpu.VMEM((CB, D), jnp.float32),
        pltpu.VMEM_SHARED((NPAD, D), jnp.float32),
        pltpu.SemaphoreType.DMA,
        pltpu.SemaphoreType.DMA,
    ],
    compiler_params=pltpu.CompilerParams(needs_layout_passes=False),
)
def _edge_kernel(tsrc_hbm, tdst_hbm, p_hbm, src_hbm, dst_hbm,
                 m_out, acc_out,
                 idx_s, idx_d, gsrc, gdst, pbuf, acc,
                 sem1, sem2):
    cid = lax.axis_index("c")
    sid = lax.axis_index("s")
    wid = sid * NC + cid
    ebase = wid * EPW

    def zrow(r, _):
        for j in range(D // 16):
            pbuf[r, pl.ds(j * 16, 16)] = jnp.zeros((16,), jnp.float32)
        return _
    lax.fori_loop(0, CB, zrow, 0)
    for k in range(ROWS_PER_TILE // CB):
        pltpu.sync_copy(pbuf, acc.at[pl.ds(sid * ROWS_PER_TILE + k * CB, CB)])
    plsc.subcore_barrier()

    def chunk_body(i, _):
        base = ebase + i * CB
        pltpu.sync_copy(src_hbm.at[pl.ds(base, CB)], idx_s)
        pltpu.sync_copy(dst_hbm.at[pl.ds(base, CB)], idx_d)
        c1 = pltpu.async_copy(tsrc_hbm.at[idx_s], gsrc, sem1)
        c2 = pltpu.async_copy(tdst_hbm.at[idx_d], gdst, sem2)
        pltpu.sync_copy(p_hbm.at[pl.ds(base, CB)], pbuf)
        c1.wait()
        c2.wait()

        def row_body(r, _):
            for j in range(D // 16):
                sl = pl.ds(j * 16, 16)
                es = gsrc[r, sl]
                ft = gsrc[r, pl.ds(D + j * 16, 16)]
                ed = gdst[r, sl]
                pv = pbuf[r, sl]
                mv = es + ed + pv
                pbuf[r, sl] = mv
                sg = 1.0 / (1.0 + jnp.exp(-mv))
                gdst[r, sl] = ft * sg
            return _
        lax.fori_loop(0, CB, row_body, 0)

        pltpu.sync_copy(pbuf, m_out.at[pl.ds(base, CB)])
        pltpu.sync_copy(gdst, acc.at[idx_d], add=True)
        return _
    lax.fori_loop(0, NCHUNK, chunk_body, 0)

    plsc.subcore_barrier()
    rbase = sid * ROWS_PER_TILE
    pltpu.sync_copy(acc.at[pl.ds(rbase, ROWS_PER_TILE)],
                    acc_out.at[cid, pl.ds(rbase, ROWS_PER_TILE)])


def _node_proj_body(x_ref, wsrc_ref, wdst_ref, bsrc_ref, bdst_ref, deg_ref,
                    tsrc_ref, tdst_ref, norms_ref):
    x = x_ref[...]
    deg = deg_ref[...]
    degsum = jnp.sum(deg, axis=0)
    norms = lax.rsqrt(jnp.maximum(degsum, 1.0))
    norms_ref[...] = norms
    es = jnp.dot(x, wsrc_ref[...], preferred_element_type=jnp.float32) + bsrc_ref[...]
    ed = jnp.dot(x, wdst_ref[...], preferred_element_type=jnp.float32) + bdst_ref[...]
    feat = x * norms[:, 0:1]
    tsrc_ref[...] = jnp.concatenate([es, feat], axis=1)
    tdst_ref[...] = ed


def _node_proj(xp, W_src, W_dst, b_src, b_dst, deg_part):
    BN = 1024
    grid = (NPAD // BN,)
    return pl.pallas_call(
        _node_proj_body,
        grid=grid,
        in_specs=[
            pl.BlockSpec((BN, D), lambda i: (i, 0)),
            pl.BlockSpec((D, D), lambda i: (0, 0)),
            pl.BlockSpec((D, D), lambda i: (0, 0)),
            pl.BlockSpec((1, D), lambda i: (0, 0)),
            pl.BlockSpec((1, D), lambda i: (0, 0)),
            pl.BlockSpec((NW, BN, 2), lambda i: (0, i, 0)),
        ],
        out_specs=[
            pl.BlockSpec((BN, 2 * D), lambda i: (i, 0)),
            pl.BlockSpec((BN, D), lambda i: (i, 0)),
            pl.BlockSpec((BN, 2), lambda i: (i, 0)),
        ],
        out_shape=[
            jax.ShapeDtypeStruct((NPAD, 2 * D), jnp.float32),
            jax.ShapeDtypeStruct((NPAD, D), jnp.float32),
            jax.ShapeDtypeStruct((NPAD, 2), jnp.float32),
        ],
    )(xp, W_src, W_dst, b_src, b_dst, deg_part)


def _edge_proj_body(ef_ref, w_ref, b_ref, p_ref):
    p_ref[...] = (jnp.dot(ef_ref[...], w_ref[...],
                          preferred_element_type=jnp.float32) + b_ref[...])


def _edge_proj(edge_feats, W_edge, b_edge):
    BE = 3200
    return pl.pallas_call(
        _edge_proj_body,
        grid=(E // BE,),
        in_specs=[
            pl.BlockSpec((BE, D), lambda i: (i, 0)),
            pl.BlockSpec((D, D), lambda i: (0, 0)),
            pl.BlockSpec((1, D), lambda i: (0, 0)),
        ],
        out_specs=pl.BlockSpec((BE, D), lambda i: (i, 0)),
        out_shape=jax.ShapeDtypeStruct((E, D), jnp.float32),
    )(edge_feats, W_edge, b_edge)


def _final_body(acc_ref, w_ref, b_ref, norms_ref, x_ref, out_ref):
    a = acc_ref[0] + acc_ref[1]
    r = jnp.dot(a, w_ref[...], preferred_element_type=jnp.float32)
    nr = norms_ref[...][:, 1:2]
    out_ref[...] = x_ref[...] + r * nr + b_ref[...]


def _final(acc_part, weight, bias, norms, node_feats):
    BF = 1000
    return pl.pallas_call(
        _final_body,
        grid=(N // BF,),
        in_specs=[
            pl.BlockSpec((NC, BF, D), lambda i: (0, i, 0)),
            pl.BlockSpec((D, D), lambda i: (0, 0)),
            pl.BlockSpec((1, D), lambda i: (0, 0)),
            pl.BlockSpec((BF, 2), lambda i: (i, 0)),
            pl.BlockSpec((BF, D), lambda i: (i, 0)),
        ],
        out_specs=pl.BlockSpec((BF, D), lambda i: (i, 0)),
        out_shape=jax.ShapeDtypeStruct((N, D), jnp.float32),
    )(acc_part, weight, bias, norms, node_feats)


def kernel(node_feats, edge_index, edge_feats, weight, bias,
           W_src, b_src, W_dst, b_dst, W_edge, b_edge):
    src = edge_index[0].astype(jnp.int32)
    dst = edge_index[1].astype(jnp.int32)

    deg_flat = _degree_kernel(src, dst)
    deg_part = deg_flat.reshape(NW, NPAD, 2)

    xp = jnp.pad(node_feats, ((0, NPAD - N), (0, 0)))
    tsrc, tdst, norms = _node_proj(
        xp, W_src, W_dst, b_src.reshape(1, D), b_dst.reshape(1, D), deg_part)

    p = _edge_proj(edge_feats, W_edge, b_edge.reshape(1, D))

    m, acc_part = _edge_kernel(tsrc, tdst, p, src, dst)

    rst = _final(acc_part, weight, bias.reshape(1, D), norms, node_feats)
    return (rst, m)


# speedup vs baseline: 1.3862x; 1.1077x over previous
import functools

import jax
import jax.numpy as jnp
from jax import lax
from jax.experimental import pallas as pl
from jax.experimental.pallas import tpu as pltpu
from jax.experimental.pallas import tpu_sc as plsc

N = 10000
E = 320000
D = 128

NC = 2
NS = 16
NW = NC * NS
NPAD = 10240
ROWS_PER_TILE = NPAD // NS
EPW = E // NW
CB = 40
NCHUNK = EPW // CB
G = 2
NSUPER = NCHUNK // G

_mesh = plsc.VectorSubcoreMesh(
    core_axis_name="c", subcore_axis_name="s", num_cores=NC, num_subcores=NS)


@functools.partial(
    pl.kernel,
    out_type=jax.ShapeDtypeStruct((NW, 2 * NPAD), jnp.float32),
    mesh=_mesh,
    scratch_types=[
        pltpu.VMEM((2 * NPAD,), jnp.float32),
        pltpu.VMEM((EPW,), jnp.int32),
        pltpu.VMEM((EPW,), jnp.int32),
    ],
    compiler_params=pltpu.CompilerParams(needs_layout_passes=False),
)
def _degree_kernel(src_hbm, dst_hbm, deg_out, hist, src_v, dst_v):
    cid = lax.axis_index("c")
    sid = lax.axis_index("s")
    wid = sid * NC + cid

    def zero_body(i, _):
        hist[pl.ds(i * 16, 16)] = jnp.zeros((16,), jnp.float32)
        return _
    lax.fori_loop(0, 2 * NPAD // 16, zero_body, 0, unroll=8)

    base = wid * EPW
    pltpu.sync_copy(src_hbm.at[pl.ds(base, EPW)], src_v)
    pltpu.sync_copy(dst_hbm.at[pl.ds(base, EPW)], dst_v)

    ones = jnp.ones((16,), jnp.float32)

    def edge_body(i, _):
        s = src_v[pl.ds(i * 16, 16)]
        d = dst_v[pl.ds(i * 16, 16)]
        plsc.addupdate_scatter(hist, [s * 2], ones)
        plsc.addupdate_scatter(hist, [d * 2 + 1], ones)
        return _
    lax.fori_loop(0, EPW // 16, edge_body, 0, unroll=4)

    pltpu.sync_copy(hist, deg_out.at[wid])


@functools.partial(
    pl.kernel,
    out_type=[
        jax.ShapeDtypeStruct((E, D), jnp.float32),
        jax.ShapeDtypeStruct((NC, NPAD, D), jnp.float32),
    ],
    mesh=_mesh,
    scratch_types=[
        pltpu.VMEM((2, G, CB), jnp.int32),
        pltpu.VMEM((2, G, CB), jnp.int32),
        pltpu.VMEM((2, CB, 2 * D), jnp.float32),
        pltpu.VMEM((2, CB, D), jnp.float32),
        pltpu.VMEM((2, CB, D), jnp.float32),
        pltpu.VMEM_SHARED((NPAD, D), jnp.float32),
        pltpu.SemaphoreType.DMA,
        pltpu.SemaphoreType.DMA,
        pltpu.SemaphoreType.DMA,
        pltpu.SemaphoreType.DMA,
        pltpu.SemaphoreType.DMA,
        pltpu.SemaphoreType.DMA,
        pltpu.SemaphoreType.DMA,
        pltpu.SemaphoreType.DMA,
    ],
    compiler_params=pltpu.CompilerParams(needs_layout_passes=False),
)
def _edge_kernel(tsrc_hbm, tdst_hbm, p_hbm, src_hbm, dst_hbm,
                 m_out, acc_out,
                 idx_s, idx_d, gsrc, gdst, pbuf, acc,
                 sts0, sts1, std0, std1, sp0, sp1, sm0, sm1):
    cid = lax.axis_index("c")
    sid = lax.axis_index("s")
    wid = sid * NC + cid
    ebase = wid * EPW
    sem_ts = (sts0, sts1)
    sem_td = (std0, std1)
    sem_p = (sp0, sp1)
    sem_m = (sm0, sm1)

    def zrow(r, _):
        for j in range(D // 16):
            pbuf[0, r, pl.ds(j * 16, 16)] = jnp.zeros((16,), jnp.float32)
        return _
    lax.fori_loop(0, CB, zrow, 0)
    for k in range(ROWS_PER_TILE // CB):
        pltpu.sync_copy(pbuf.at[0],
                        acc.at[pl.ds(sid * ROWS_PER_TILE + k * CB, CB)])
    plsc.subcore_barrier()

    def load_idx(s, u):
        pltpu.sync_copy(src_hbm.at[wid, s], idx_s.at[u])
        pltpu.sync_copy(dst_hbm.at[wid, s], idx_d.at[u])

    def issue_tsrc(u, g, b):
        pltpu.async_copy(tsrc_hbm.at[idx_s.at[u, g]], gsrc.at[b], sem_ts[b])

    def issue_tdst_p(i, u, g, b):
        pltpu.async_copy(tdst_hbm.at[idx_d.at[u, g]], gdst.at[b], sem_td[b])
        pltpu.async_copy(p_hbm.at[pl.ds(ebase + i * CB, CB)], pbuf.at[b],
                         sem_p[b])

    def chunk_body(s, u, g, b, issue_next):
        i = s * G + g
        base = ebase + i * CB
        pltpu.make_async_copy(tsrc_hbm.at[idx_s.at[u, g]], gsrc.at[b],
                              sem_ts[b]).wait()
        pltpu.make_async_copy(tdst_hbm.at[idx_d.at[u, g]], gdst.at[b],
                              sem_td[b]).wait()
        pltpu.make_async_copy(p_hbm.at[pl.ds(base, CB)], pbuf.at[b],
                              sem_p[b]).wait()

        def row_body(r, _):
            for j in range(D // 16):
                sl = pl.ds(j * 16, 16)
                es = gsrc[b, r, sl]
                ft = gsrc[b, r, pl.ds(D + j * 16, 16)]
                ed = gdst[b, r, sl]
                pv = pbuf[b, r, sl]
                mv = es + ed + pv
                pbuf[b, r, sl] = mv
                sg = 1.0 / (1.0 + jnp.exp(-mv))
                gdst[b, r, sl] = ft * sg
            return _
        lax.fori_loop(0, CB, row_body, 0, unroll=2)

        if g + 2 < G:
            nu, ng = u, g + 2
        else:
            nu, ng = 1 - u, g + 2 - G
        if issue_next:
            issue_tsrc(nu, ng, b)
        pltpu.async_copy(pbuf.at[b], m_out.at[pl.ds(base, CB)], sem_m[b])
        pltpu.sync_copy(gdst.at[b], acc.at[idx_d.at[u, g]], add=True)
        pltpu.make_async_copy(pbuf.at[b], m_out.at[pl.ds(base, CB)],
                              sem_m[b]).wait()
        if issue_next:
            issue_tdst_p(i + 2, nu, ng, b)

    def super_body(s, u, prefetch_next, issue_last):
        if prefetch_next:
            load_idx(s + 1, 1 - u)
        for g in range(G):
            chunk_body(s, u, g, g % 2, issue_last or g + 2 < G)

    load_idx(0, 0)
    issue_tsrc(0, 0, 0)
    issue_tdst_p(0, 0, 0, 0)
    issue_tsrc(0, 1, 1)
    issue_tdst_p(1, 0, 1, 1)

    def pair_body(k, _):
        super_body(2 * k, 0, True, True)
        super_body(2 * k + 1, 1, True, True)
        return _
    lax.fori_loop(0, (NSUPER - 1) // 2, pair_body, 0)
    super_body(NSUPER - 1, 0, False, False)

    plsc.subcore_barrier()
    rbase = sid * ROWS_PER_TILE
    pltpu.sync_copy(acc.at[pl.ds(rbase, ROWS_PER_TILE)],
                    acc_out.at[cid, pl.ds(rbase, ROWS_PER_TILE)])


def _node_proj_body(x_ref, wsrc_ref, wdst_ref, bsrc_ref, bdst_ref, deg_ref,
                    tsrc_ref, tdst_ref, norms_ref):
    x = x_ref[...]
    deg = deg_ref[...]
    degsum = jnp.sum(deg, axis=0)
    norms = lax.rsqrt(jnp.maximum(degsum, 1.0))
    norms_ref[...] = norms
    es = jnp.dot(x, wsrc_ref[...], preferred_element_type=jnp.float32) + bsrc_ref[...]
    ed = jnp.dot(x, wdst_ref[...], preferred_element_type=jnp.float32) + bdst_ref[...]
    feat = x * norms[:, 0:1]
    tsrc_ref[...] = jnp.concatenate([es, feat], axis=1)
    tdst_ref[...] = ed


def _node_proj(xp, W_src, W_dst, b_src, b_dst, deg_part):
    BN = 1024
    grid = (NPAD // BN,)
    return pl.pallas_call(
        _node_proj_body,
        grid=grid,
        in_specs=[
            pl.BlockSpec((BN, D), lambda i: (i, 0)),
            pl.BlockSpec((D, D), lambda i: (0, 0)),
            pl.BlockSpec((D, D), lambda i: (0, 0)),
            pl.BlockSpec((1, D), lambda i: (0, 0)),
            pl.BlockSpec((1, D), lambda i: (0, 0)),
            pl.BlockSpec((NW, BN, 2), lambda i: (0, i, 0)),
        ],
        out_specs=[
            pl.BlockSpec((BN, 2 * D), lambda i: (i, 0)),
            pl.BlockSpec((BN, D), lambda i: (i, 0)),
            pl.BlockSpec((BN, 2), lambda i: (i, 0)),
        ],
        out_shape=[
            jax.ShapeDtypeStruct((NPAD, 2 * D), jnp.float32),
            jax.ShapeDtypeStruct((NPAD, D), jnp.float32),
            jax.ShapeDtypeStruct((NPAD, 2), jnp.float32),
        ],
    )(xp, W_src, W_dst, b_src, b_dst, deg_part)


def _edge_proj_body(ef_ref, w_ref, b_ref, p_ref):
    p_ref[...] = (jnp.dot(ef_ref[...], w_ref[...],
                          preferred_element_type=jnp.float32) + b_ref[...])


def _edge_proj(edge_feats, W_edge, b_edge):
    BE = 3200
    return pl.pallas_call(
        _edge_proj_body,
        grid=(E // BE,),
        in_specs=[
            pl.BlockSpec((BE, D), lambda i: (i, 0)),
            pl.BlockSpec((D, D), lambda i: (0, 0)),
            pl.BlockSpec((1, D), lambda i: (0, 0)),
        ],
        out_specs=pl.BlockSpec((BE, D), lambda i: (i, 0)),
        out_shape=jax.ShapeDtypeStruct((E, D), jnp.float32),
    )(edge_feats, W_edge, b_edge)


def _final_body(acc_ref, w_ref, b_ref, norms_ref, x_ref, out_ref):
    a = acc_ref[0] + acc_ref[1]
    r = jnp.dot(a, w_ref[...], preferred_element_type=jnp.float32)
    nr = norms_ref[...][:, 1:2]
    out_ref[...] = x_ref[...] + r * nr + b_ref[...]


def _final(acc_part, weight, bias, norms, node_feats):
    BF = 1000
    return pl.pallas_call(
        _final_body,
        grid=(N // BF,),
        in_specs=[
            pl.BlockSpec((NC, BF, D), lambda i: (0, i, 0)),
            pl.BlockSpec((D, D), lambda i: (0, 0)),
            pl.BlockSpec((1, D), lambda i: (0, 0)),
            pl.BlockSpec((BF, 2), lambda i: (i, 0)),
            pl.BlockSpec((BF, D), lambda i: (i, 0)),
        ],
        out_specs=pl.BlockSpec((BF, D), lambda i: (i, 0)),
        out_shape=jax.ShapeDtypeStruct((N, D), jnp.float32),
    )(acc_part, weight, bias, norms, node_feats)


def kernel(node_feats, edge_index, edge_feats, weight, bias,
           W_src, b_src, W_dst, b_dst, W_edge, b_edge):
    src = edge_index[0].astype(jnp.int32)
    dst = edge_index[1].astype(jnp.int32)

    deg_flat = _degree_kernel(src, dst)
    deg_part = deg_flat.reshape(NW, NPAD, 2)

    xp = jnp.pad(node_feats, ((0, NPAD - N), (0, 0)))
    tsrc, tdst, norms = _node_proj(
        xp, W_src, W_dst, b_src.reshape(1, D), b_dst.reshape(1, D), deg_part)

    p = _edge_proj(edge_feats, W_edge, b_edge.reshape(1, D))

    src4 = src.reshape(NW, NSUPER, G, CB)
    dst4 = dst.reshape(NW, NSUPER, G, CB)
    m, acc_part = _edge_kernel(tsrc, tdst, p, src4, dst4)

    rst = _final(acc_part, weight, bias.reshape(1, D), norms, node_feats)
    return (rst, m)
